# baseline (speedup 1.0000x reference)
import jax
import jax.numpy as jnp
from jax.experimental import pallas as pl
from jax.experimental.pallas import tpu as pltpu

def _tree0(x, op):
    r = x.shape[0]
    while r > 1:
        h = r // 2
        y = op(x[:h], x[h:2 * h])
        if r % 2:
            y = jnp.concatenate([y, x[2 * h:r]], axis=0)
        x = y
        r = x.shape[0]
    return x


NMS_TOPN = 300
SEL_ROWS = 304
MROWS = 3
TOTAL_POS = 64
TOTAL_NEG = 64
IOU_THR = 0.5


def _nms_roi_kernel(ay1, ax1, ay2, ax2, sc, gy1, gx1, gy2, gx2,
                    roi_out, gt_out, sel_ref):
    rows_n, b, _ = ay1.shape
    n = rows_n * 128
    y1 = ay1[...]
    x1 = ax1[...]
    y2 = ay2[...]
    x2 = ax2[...]
    area = jnp.maximum(y2 - y1, 0.0) * jnp.maximum(x2 - x1, 0.0)
    lane = jax.lax.broadcasted_iota(jnp.int32, (1, 1, 128), 2)
    flat = (jax.lax.broadcasted_iota(jnp.int32, (rows_n, b, 128), 0) * 128
            + jax.lax.broadcasted_iota(jnp.int32, (rows_n, b, 128), 2))
    flat_m = (jax.lax.broadcasted_iota(jnp.int32, (MROWS, b, 128), 0) * 128
              + jax.lax.broadcasted_iota(jnp.int32, (MROWS, b, 128), 2))
    g1 = gy1[...]
    g2 = gx1[...]
    g3 = gy2[...]
    g4 = gx2[...]
    ag = jnp.maximum(g3 - g1, 0.0) * jnp.maximum(g4 - g2, 0.0)

    def nms_body(i, state):
        live, mT, bT = state
        m = jnp.max(_tree0(live, jnp.maximum), axis=2, keepdims=True)
        cand = jnp.where(live == m, flat, n)
        c, t = cand, (y1, x1, y2, x2)
        r = rows_n
        while r > 1:
            h = r // 2
            take = c[:h] <= c[h:2 * h]
            c2 = jnp.where(take, c[:h], c[h:2 * h])
            t2 = tuple(jnp.where(take, u[:h], u[h:2 * h]) for u in t)
            if r % 2:
                c2 = jnp.concatenate([c2, c[2 * h:r]], axis=0)
                t2 = tuple(jnp.concatenate([u2, u[2 * h:r]], axis=0)
                           for u2, u in zip(t2, t))
            c, t = c2, t2
            r = c.shape[0]
        idx = jnp.min(c, axis=2, keepdims=True)
        lmask = c == idx
        msk = flat == idx

        def pick(arr):
            return jnp.sum(jnp.where(lmask, arr, 0.0), axis=2, keepdims=True)

        by1 = pick(t[0])
        bx1 = pick(t[1])
        by2 = pick(t[2])
        bx2 = pick(t[3])
        ba = jnp.maximum(by2 - by1, 0.0) * jnp.maximum(bx2 - bx1, 0.0)
        yy1 = jnp.maximum(by1, y1)
        xx1 = jnp.maximum(bx1, x1)
        yy2 = jnp.minimum(by2, y2)
        xx2 = jnp.minimum(bx2, x2)
        inter = jnp.maximum(yy2 - yy1, 0.0) * jnp.maximum(xx2 - xx1, 0.0)
        iou = inter / (ba + area - inter + 1e-8)
        live = jnp.where(jnp.logical_or(iou > IOU_THR, msk), -1.0, live)
        vld = m > 0.0
        bz1 = jnp.where(vld, by1, 0.0)
        bz2 = jnp.where(vld, bx1, 0.0)
        bz3 = jnp.where(vld, by2, 0.0)
        bz4 = jnp.where(vld, bx2, 0.0)
        row = (jnp.where(lane == 0, bz1, 0.0)
               + jnp.where(lane == 1, bz2, 0.0)
               + jnp.where(lane == 2, bz3, 0.0)
               + jnp.where(lane == 3, bz4, 0.0))
        sel_ref[pl.ds(i, 1), :, :] = row

        gyy1 = jnp.maximum(bz1, g1)
        gxx1 = jnp.maximum(bz2, g2)
        gyy2 = jnp.minimum(bz3, g3)
        gxx2 = jnp.minimum(bz4, g4)
        ginter = (jnp.maximum(gyy2 - gyy1, 0.0)
                  * jnp.maximum(gxx2 - gxx1, 0.0))
        ab = jnp.maximum(bz3 - bz1, 0.0) * jnp.maximum(bz4 - bz2, 0.0)
        giou = ginter / (ab + ag - ginter + 1e-8)
        mi = jnp.max(giou, axis=2, keepdims=True)
        bi = jnp.min(jnp.where(giou == mi, lane, 128), axis=2, keepdims=True)
        at_i = flat_m == i
        mT = jnp.where(at_i, mi, mT)
        bT = jnp.where(at_i, bi, bT)
        return (live, mT, bT)

    state0 = (sc[...],
              jnp.full((MROWS, b, 128), -1.0, jnp.float32),
              jnp.zeros((MROWS, b, 128), jnp.int32))
    _, mT, bT = jax.lax.fori_loop(0, NMS_TOPN, nms_body, state0, unroll=50)
    sel_ref[NMS_TOPN:SEL_ROWS, :, :] = jnp.zeros(
        (SEL_ROWS - NMS_TOPN, b, 128), jnp.float32)
    sel = sel_ref[...]

    rows304 = jax.lax.broadcasted_iota(jnp.int32, (SEL_ROWS, b, 1), 0)
    l64 = jax.lax.broadcasted_iota(jnp.int32, (1, b, TOTAL_POS), 2)

    def top_body(j, st):
        mg, gacc = st
        m = jnp.max(jnp.max(mg, axis=0, keepdims=True), axis=2,
                    keepdims=True)
        cand = jnp.where(mg == m, flat_m, SEL_ROWS)
        idx = jnp.min(jnp.min(cand, axis=0, keepdims=True), axis=2,
                      keepdims=True)
        at = flat_m == idx
        gv = jnp.sum(jnp.sum(jnp.where(at, bT, 0), axis=0, keepdims=True),
                     axis=2, keepdims=True)
        gacc = gacc + jnp.where(l64 == j, gv, 0)
        er = rows304 == idx
        boxrow = jnp.sum(jnp.where(er, sel, 0.0), axis=0, keepdims=True)
        roi_out[pl.ds(j, 1), :, :] = boxrow
        mg = jnp.where(at, -2.0, mg)
        return (mg, gacc)

    _, gacc = jax.lax.fori_loop(
        0, TOTAL_POS, top_body,
        (mT, jnp.zeros((1, b, TOTAL_POS), jnp.int32)), unroll=8)
    gt_out[...] = gacc[0]


def kernel(rpn_bbox_deltas, rpn_labels, anchors, gt_boxes):
    b, n = anchors.shape[0], anchors.shape[1]
    rows_n = n // 128
    g = gt_boxes.shape[1]
    deltas = rpn_bbox_deltas.reshape(b, n, 4)
    scores = rpn_labels.reshape(b, n)

    aw = anchors[..., 3] - anchors[..., 1]
    ah = anchors[..., 2] - anchors[..., 0]
    acx = anchors[..., 1] + 0.5 * aw
    acy = anchors[..., 0] + 0.5 * ah
    bw = jnp.exp(deltas[..., 3]) * aw
    bh = jnp.exp(deltas[..., 2]) * ah
    bcx = deltas[..., 1] * aw + acx
    bcy = deltas[..., 0] * ah + acy
    y1 = bcy - 0.5 * bh
    x1 = bcx - 0.5 * bw
    y2 = bh + y1
    x2 = bw + x1

    def pack(a):
        return jnp.transpose(a.reshape(b, rows_n, 128), (1, 0, 2))

    gplanes = []
    for k in range(4):
        gp = jnp.zeros((b, 128), jnp.float32).at[:, :g].set(gt_boxes[:, :, k])
        gplanes.append(gp[None])

    out_shape = (jax.ShapeDtypeStruct((TOTAL_POS, b, 128), jnp.float32),
                 jax.ShapeDtypeStruct((b, TOTAL_POS), jnp.int32))
    roi_pack, gt_idx = pl.pallas_call(
        _nms_roi_kernel,
        out_shape=out_shape,
        scratch_shapes=[pltpu.VMEM((SEL_ROWS, b, 128), jnp.float32)],
    )(pack(y1), pack(x1), pack(y2), pack(x2), pack(scores), *gplanes)

    roi = jnp.transpose(roi_pack, (1, 0, 2))[:, :, :4]
    roi_bboxes = jnp.concatenate(
        [roi, jnp.zeros((b, TOTAL_NEG, 4), jnp.float32)], axis=1)
    return roi_bboxes, gt_idx


# speedup vs baseline: 1.0052x; 1.0052x over previous
import jax
import jax.numpy as jnp
from jax.experimental import pallas as pl
from jax.experimental.pallas import tpu as pltpu

def _tree0(x, op):
    r = x.shape[0]
    while r > 1:
        h = r // 2
        y = op(x[:h], x[h:2 * h])
        if r % 2:
            y = jnp.concatenate([y, x[2 * h:r]], axis=0)
        x = y
        r = x.shape[0]
    return x


NMS_TOPN = 300
SEL_ROWS = 304
MROWS = 3
TOTAL_POS = 64
TOTAL_NEG = 64
IOU_THR = 0.5


def _nms_roi_kernel(ay1, ax1, ay2, ax2, sc, gy1, gx1, gy2, gx2,
                    roi_out, gt_out, sel_ref):
    rows_n, b, _ = ay1.shape
    n = rows_n * 128
    y1 = ay1[...]
    x1 = ax1[...]
    y2 = ay2[...]
    x2 = ax2[...]
    area = jnp.maximum(y2 - y1, 0.0) * jnp.maximum(x2 - x1, 0.0)
    lane = jax.lax.broadcasted_iota(jnp.int32, (1, 1, 128), 2)
    flat = (jax.lax.broadcasted_iota(jnp.int32, (rows_n, b, 128), 0) * 128
            + jax.lax.broadcasted_iota(jnp.int32, (rows_n, b, 128), 2))
    flat_m = (jax.lax.broadcasted_iota(jnp.int32, (MROWS, b, 128), 0) * 128
              + jax.lax.broadcasted_iota(jnp.int32, (MROWS, b, 128), 2))
    g1 = gy1[...]
    g2 = gx1[...]
    g3 = gy2[...]
    g4 = gx2[...]
    ag = jnp.maximum(g3 - g1, 0.0) * jnp.maximum(g4 - g2, 0.0)

    def nms_body(i, state):
        live, mT, bT = state
        m = jnp.max(_tree0(live, jnp.maximum), axis=2, keepdims=True)
        cand = jnp.where(live == m, flat, n)
        c, t = cand, (y1, x1, y2, x2)
        r = rows_n
        while r > 1:
            h = r // 2
            take = c[:h] <= c[h:2 * h]
            c2 = jnp.where(take, c[:h], c[h:2 * h])
            t2 = tuple(jnp.where(take, u[:h], u[h:2 * h]) for u in t)
            if r % 2:
                c2 = jnp.concatenate([c2, c[2 * h:r]], axis=0)
                t2 = tuple(jnp.concatenate([u2, u[2 * h:r]], axis=0)
                           for u2, u in zip(t2, t))
            c, t = c2, t2
            r = c.shape[0]
        idx = jnp.min(c, axis=2, keepdims=True)
        lmask = c == idx
        msk = flat == idx

        def pick(arr):
            return jnp.sum(jnp.where(lmask, arr, 0.0), axis=2, keepdims=True)

        by1 = pick(t[0])
        bx1 = pick(t[1])
        by2 = pick(t[2])
        bx2 = pick(t[3])
        ba = jnp.maximum(by2 - by1, 0.0) * jnp.maximum(bx2 - bx1, 0.0)
        yy1 = jnp.maximum(by1, y1)
        xx1 = jnp.maximum(bx1, x1)
        yy2 = jnp.minimum(by2, y2)
        xx2 = jnp.minimum(bx2, x2)
        inter = jnp.maximum(yy2 - yy1, 0.0) * jnp.maximum(xx2 - xx1, 0.0)
        iou = inter / (ba + area - inter + 1e-8)
        live = jnp.where(jnp.logical_or(iou > IOU_THR, msk), -1.0, live)
        vld = m > 0.0
        bz1 = jnp.where(vld, by1, 0.0)
        bz2 = jnp.where(vld, bx1, 0.0)
        bz3 = jnp.where(vld, by2, 0.0)
        bz4 = jnp.where(vld, bx2, 0.0)
        row = (jnp.where(lane == 0, bz1, 0.0)
               + jnp.where(lane == 1, bz2, 0.0)
               + jnp.where(lane == 2, bz3, 0.0)
               + jnp.where(lane == 3, bz4, 0.0))
        sel_ref[pl.ds(i, 1), :, :] = row

        gyy1 = jnp.maximum(bz1, g1)
        gxx1 = jnp.maximum(bz2, g2)
        gyy2 = jnp.minimum(bz3, g3)
        gxx2 = jnp.minimum(bz4, g4)
        ginter = (jnp.maximum(gyy2 - gyy1, 0.0)
                  * jnp.maximum(gxx2 - gxx1, 0.0))
        ab = jnp.maximum(bz3 - bz1, 0.0) * jnp.maximum(bz4 - bz2, 0.0)
        giou = ginter / (ab + ag - ginter + 1e-8)
        mi = jnp.max(giou, axis=2, keepdims=True)
        bi = jnp.min(jnp.where(giou == mi, lane, 128), axis=2, keepdims=True)
        at_i = flat_m == i
        mT = jnp.where(at_i, mi, mT)
        bT = jnp.where(at_i, bi, bT)
        return (live, mT, bT)

    state0 = (sc[...],
              jnp.full((MROWS, b, 128), -1.0, jnp.float32),
              jnp.zeros((MROWS, b, 128), jnp.int32))
    _, mT, bT = jax.lax.fori_loop(0, NMS_TOPN, nms_body, state0, unroll=25)
    sel_ref[NMS_TOPN:SEL_ROWS, :, :] = jnp.zeros(
        (SEL_ROWS - NMS_TOPN, b, 128), jnp.float32)
    sel = sel_ref[...]

    rows304 = jax.lax.broadcasted_iota(jnp.int32, (SEL_ROWS, b, 1), 0)
    l64 = jax.lax.broadcasted_iota(jnp.int32, (1, b, TOTAL_POS), 2)

    def top_body(j, st):
        mg, gacc = st
        m = jnp.max(jnp.max(mg, axis=0, keepdims=True), axis=2,
                    keepdims=True)
        cand = jnp.where(mg == m, flat_m, SEL_ROWS)
        idx = jnp.min(jnp.min(cand, axis=0, keepdims=True), axis=2,
                      keepdims=True)
        at = flat_m == idx
        gv = jnp.sum(jnp.sum(jnp.where(at, bT, 0), axis=0, keepdims=True),
                     axis=2, keepdims=True)
        gacc = gacc + jnp.where(l64 == j, gv, 0)
        er = rows304 == idx
        boxrow = jnp.sum(jnp.where(er, sel, 0.0), axis=0, keepdims=True)
        roi_out[pl.ds(j, 1), :, :] = boxrow
        mg = jnp.where(at, -2.0, mg)
        return (mg, gacc)

    _, gacc = jax.lax.fori_loop(
        0, TOTAL_POS, top_body,
        (mT, jnp.zeros((1, b, TOTAL_POS), jnp.int32)), unroll=8)
    gt_out[...] = gacc[0]


def kernel(rpn_bbox_deltas, rpn_labels, anchors, gt_boxes):
    b, n = anchors.shape[0], anchors.shape[1]
    rows_n = n // 128
    g = gt_boxes.shape[1]
    deltas = rpn_bbox_deltas.reshape(b, n, 4)
    scores = rpn_labels.reshape(b, n)

    aw = anchors[..., 3] - anchors[..., 1]
    ah = anchors[..., 2] - anchors[..., 0]
    acx = anchors[..., 1] + 0.5 * aw
    acy = anchors[..., 0] + 0.5 * ah
    bw = jnp.exp(deltas[..., 3]) * aw
    bh = jnp.exp(deltas[..., 2]) * ah
    bcx = deltas[..., 1] * aw + acx
    bcy = deltas[..., 0] * ah + acy
    y1 = bcy - 0.5 * bh
    x1 = bcx - 0.5 * bw
    y2 = bh + y1
    x2 = bw + x1

    def pack(a):
        return jnp.transpose(a.reshape(b, rows_n, 128), (1, 0, 2))

    gplanes = []
    for k in range(4):
        gp = jnp.zeros((b, 128), jnp.float32).at[:, :g].set(gt_boxes[:, :, k])
        gplanes.append(gp[None])

    out_shape = (jax.ShapeDtypeStruct((TOTAL_POS, b, 128), jnp.float32),
                 jax.ShapeDtypeStruct((b, TOTAL_POS), jnp.int32))
    roi_pack, gt_idx = pl.pallas_call(
        _nms_roi_kernel,
        out_shape=out_shape,
        scratch_shapes=[pltpu.VMEM((SEL_ROWS, b, 128), jnp.float32)],
    )(pack(y1), pack(x1), pack(y2), pack(x2), pack(scores), *gplanes)

    roi = jnp.transpose(roi_pack, (1, 0, 2))[:, :, :4]
    roi_bboxes = jnp.concatenate(
        [roi, jnp.zeros((b, TOTAL_NEG, 4), jnp.float32)], axis=1)
    return roi_bboxes, gt_idx
